# baseline (speedup 1.0000x reference)
import functools

import jax
import jax.numpy as jnp
from jax import lax
from jax.experimental import pallas as pl
from jax.experimental.pallas import tpu as pltpu
from jax.experimental.pallas import tpu_sc as plsc

NBINS = 512
L = 16
NB2 = NBINS + L
NW = 32
B, C, H, W = 16, 96, 224, 224
ROWLEN = H * W
NROWS = B * C
ROWS_PER_W = NROWS // NW
CHUNKS = ROWLEN // L
HCHUNKS = NBINS // L
UNROLL = 8


def _sc_body(K, x_hbm, params_hbm, out_hbm, bufa, bufb, cnt, outbuf, pv,
             sem0, sem1):
    wid = lax.axis_index("s") * 2 + lax.axis_index("c")
    base = wid * ROWS_PER_W

    pltpu.sync_copy(params_hbm, pv)
    pvec = pv[pl.ds(0, L)]
    lam = pvec[0]
    inv1md = pvec[1]
    invw = pvec[2]
    kf = jnp.float32(K)

    lanes = lax.broadcasted_iota(jnp.int32, (L,), 0)
    lanebase = lanes * NB2
    ones = jnp.full((L,), 1.0, dtype=jnp.float32)
    zeros = jnp.zeros((L,), dtype=jnp.float32)
    descoff = jnp.float32(L - 1) - lanes.astype(jnp.float32)
    magic = jnp.float32(12582912.0)
    bias = jnp.int32(0x4B400000)

    @plsc.parallel_loop(0, L * NB2, step=UNROLL * L, unroll=2)
    def zero_hist(i):
        for u in range(UNROLL):
            cnt[pl.ds(i + u * L, L)] = zeros

    def row_compute(buf):
        first = buf[0, pl.ds(0, L)]

        @plsc.parallel_loop(0, H, unroll=2,
                            carry=((first,) * 7, (first,) * 7))
        def mm_loop(i, carry):
            mns, mxs = carry
            vs = [buf[i, pl.ds(u * L, L)] for u in range(14)]
            mns = tuple(jnp.minimum(m, jnp.minimum(vs[2 * j], vs[2 * j + 1]))
                        for j, m in enumerate(mns))
            mxs = tuple(jnp.maximum(m, jnp.maximum(vs[2 * j], vs[2 * j + 1]))
                        for j, m in enumerate(mxs))
            return mns, mxs

        mns, mxs = mm_loop
        mn, mx = mns[0], mxs[0]
        for u in range(1, 7):
            mn = jnp.minimum(mn, mns[u])
            mx = jnp.maximum(mx, mxs[u])
        lo = -jnp.max(-mn)
        hi = jnp.max(mx)
        rng = jnp.maximum(hi - lo, jnp.float32(1e-30))
        scale = jnp.full((L,), jnp.float32(NBINS)) / rng
        bw = rng * jnp.float32(1.0 / NBINS)
        adj = magic + lanebase.astype(jnp.float32)

        @plsc.parallel_loop(0, H, unroll=4)
        def sc_loop(i):
            for u in range(14):
                v = buf[i, pl.ds(u * L, L)]
                t = (v - lo) * scale + adj
                idx = plsc.bitcast(t, jnp.int32) - bias
                plsc.addupdate_scatter(cnt, [idx], ones)

        ov = cnt[pl.ds(NBINS, L)]
        cnt[pl.ds(NBINS, L)] = jnp.zeros((L,), jnp.float32)
        for l in range(1, L):
            off = l * NB2 + NBINS
            ov = ov + cnt[pl.ds(off, L)]
            cnt[pl.ds(off, L)] = jnp.zeros((L,), jnp.float32)
        ov0 = ov[0]
        top = cnt[pl.ds(NBINS - L, L)]
        cnt[pl.ds(NBINS - L, L)] = top + jnp.where(
            lanes == L - 1, ov0, 0.0)

        @plsc.parallel_loop(0, HCHUNKS, unroll=2,
                            carry=(zeros, jnp.float32(0.0)))
        def rd_loop(j, carry):
            acc, rank = carry
            c0 = (HCHUNKS - 1 - j) * L
            cc = cnt[pl.ds(c0, L)]
            cnt[pl.ds(c0, L)] = zeros
            for l in range(1, L):
                off = l * NB2 + c0
                cc = cc + cnt[pl.ds(off, L)]
                cnt[pl.ds(off, L)] = zeros
            rc = lax.rev(cc, (0,))
            incl = plsc.cumsum(rc)
            r_excl = incl - rc + rank
            m = jnp.clip(kf - r_excl, 0.0, rc)
            om = (jnp.exp(lam * r_excl) - jnp.exp(lam * (r_excl + m))) * inv1md
            val = lo + (c0.astype(jnp.float32) + descoff) * bw
            return acc + om * val, rank + incl[L - 1]

        acc, _ = rd_loop
        return jnp.sum(acc) * invw

    pltpu.async_copy(x_hbm.at[base], bufa, sem0)

    def pair_body(p, resvec):
        row0 = base + 2 * p
        pltpu.async_copy(x_hbm.at[row0 + 1], bufb, sem1)
        pltpu.make_async_copy(x_hbm.at[row0], bufa, sem0).wait()
        s0 = row_compute(bufa)

        @pl.when(2 * p + 2 < ROWS_PER_W)
        def _():
            pltpu.async_copy(x_hbm.at[row0 + 2], bufa, sem0)

        pltpu.make_async_copy(x_hbm.at[row0 + 1], bufb, sem1).wait()
        s1 = row_compute(bufb)

        r0 = (2 * p) % L
        resvec = (resvec + jnp.where(lanes == r0, s0, 0.0)
                  + jnp.where(lanes == r0 + 1, s1, 0.0))

        @pl.when((p % (L // 2)) == (L // 2 - 1))
        def _():
            outbuf[pl.ds((p - (L // 2 - 1)) * 2, L)] = resvec

        return jnp.where(p % (L // 2) == (L // 2 - 1),
                         jnp.zeros((L,), jnp.float32), resvec)

    lax.fori_loop(0, ROWS_PER_W // 2, pair_body,
                  jnp.zeros((L,), jnp.float32))
    pltpu.sync_copy(outbuf, out_hbm.at[pl.ds(base, ROWS_PER_W)])


def kernel(input, gwrp_w):
    x = input.reshape(NROWS, H, W)
    K = gwrp_w.shape[0]
    d = gwrp_w[1]
    lam = jnp.log(d)
    inv1md = 1.0 / (1.0 - d)
    invw = 1.0 / jnp.sum(gwrp_w)
    params = jnp.zeros((L,), jnp.float32)
    params = params.at[0].set(lam).at[1].set(inv1md).at[2].set(invw)

    mesh = plsc.VectorSubcoreMesh(core_axis_name="c", subcore_axis_name="s")
    run = pl.kernel(
        functools.partial(_sc_body, K),
        out_type=jax.ShapeDtypeStruct((NROWS,), jnp.float32),
        mesh=mesh,
        compiler_params=pltpu.CompilerParams(needs_layout_passes=False,
                                             use_tc_tiling_on_sc=True),
        scratch_types=[
            pltpu.VMEM((H, W), jnp.float32),
            pltpu.VMEM((H, W), jnp.float32),
            pltpu.VMEM((L * NB2,), jnp.float32),
            pltpu.VMEM((ROWS_PER_W,), jnp.float32),
            pltpu.VMEM((L,), jnp.float32),
            pltpu.SemaphoreType.DMA,
            pltpu.SemaphoreType.DMA,
        ],
    )
    out = run(x, params)
    return out.reshape(B, C)


# speedup vs baseline: 1.0259x; 1.0259x over previous
import functools

import jax
import jax.numpy as jnp
from jax import lax
from jax.experimental import pallas as pl
from jax.experimental.pallas import tpu as pltpu
from jax.experimental.pallas import tpu_sc as plsc

NBINS = 512
L = 16
NB2 = NBINS + L
NW = 32
B, C, H, W = 16, 96, 224, 224
ROWLEN = H * W
NROWS = B * C
ROWS_PER_W = NROWS // NW
CHUNKS = ROWLEN // L
HCHUNKS = NBINS // L
UNROLL = 8


def _sc_body(K, x_hbm, params_hbm, out_hbm, bufa, bufb, cnt, outbuf, pv,
             sem0, sem1):
    wid = lax.axis_index("s") * 2 + lax.axis_index("c")
    base = wid * ROWS_PER_W

    pltpu.sync_copy(params_hbm, pv)
    pvec = pv[pl.ds(0, L)]
    lam = pvec[0]
    inv1md = pvec[1]
    invw = pvec[2]
    kf = jnp.float32(K)

    lanes = lax.broadcasted_iota(jnp.int32, (L,), 0)
    lanebase = lanes * NB2
    ones = jnp.full((L,), 1.0, dtype=jnp.float32)
    zeros = jnp.zeros((L,), dtype=jnp.float32)
    descoff = jnp.float32(L - 1) - lanes.astype(jnp.float32)
    magic = jnp.float32(12582912.0)
    bias = jnp.int32(0x4B400000)

    @plsc.parallel_loop(0, L * NB2, step=UNROLL * L, unroll=2)
    def zero_hist(i):
        for u in range(UNROLL):
            cnt[pl.ds(i + u * L, L)] = zeros

    def row_compute(buf):
        first = buf[0, pl.ds(0, L)]

        @plsc.parallel_loop(0, H, unroll=2,
                            carry=((first,) * 7, (first,) * 7))
        def mm_loop(i, carry):
            mns, mxs = carry
            vs = [buf[i, pl.ds(u * L, L)] for u in range(14)]
            mns = tuple(jnp.minimum(m, jnp.minimum(vs[2 * j], vs[2 * j + 1]))
                        for j, m in enumerate(mns))
            mxs = tuple(jnp.maximum(m, jnp.maximum(vs[2 * j], vs[2 * j + 1]))
                        for j, m in enumerate(mxs))
            return mns, mxs

        mns, mxs = mm_loop
        mn, mx = mns[0], mxs[0]
        for u in range(1, 7):
            mn = jnp.minimum(mn, mns[u])
            mx = jnp.maximum(mx, mxs[u])
        lo = -jnp.max(-mn)
        hi = jnp.max(mx)
        rng = jnp.maximum(hi - lo, jnp.float32(1e-30))
        scale = jnp.full((L,), jnp.float32(NBINS)) / rng
        bw = rng * jnp.float32(1.0 / NBINS)
        adj = magic + lanebase.astype(jnp.float32)

        @plsc.parallel_loop(0, H, unroll=2)
        def sc_loop(i):
            vs = [buf[i, pl.ds(u * L, L)] for u in range(14)]
            ts = [(v - lo) * scale + adj for v in vs]
            idxs = [plsc.bitcast(t, jnp.int32) - bias for t in ts]
            for idx in idxs:
                plsc.addupdate_scatter(cnt, [idx], ones)

        ov = cnt[pl.ds(NBINS, L)]
        cnt[pl.ds(NBINS, L)] = jnp.zeros((L,), jnp.float32)
        for l in range(1, L):
            off = l * NB2 + NBINS
            ov = ov + cnt[pl.ds(off, L)]
            cnt[pl.ds(off, L)] = jnp.zeros((L,), jnp.float32)
        ov0 = ov[0]
        top = cnt[pl.ds(NBINS - L, L)]
        cnt[pl.ds(NBINS - L, L)] = top + jnp.where(
            lanes == L - 1, ov0, 0.0)

        @plsc.parallel_loop(0, HCHUNKS, unroll=2,
                            carry=(zeros, jnp.float32(0.0)))
        def rd_loop(j, carry):
            acc, rank = carry
            c0 = (HCHUNKS - 1 - j) * L
            cc = cnt[pl.ds(c0, L)]
            cnt[pl.ds(c0, L)] = zeros
            for l in range(1, L):
                off = l * NB2 + c0
                cc = cc + cnt[pl.ds(off, L)]
                cnt[pl.ds(off, L)] = zeros
            rc = lax.rev(cc, (0,))
            incl = plsc.cumsum(rc)
            r_excl = incl - rc + rank
            m = jnp.clip(kf - r_excl, 0.0, rc)
            om = (jnp.exp(lam * r_excl) - jnp.exp(lam * (r_excl + m))) * inv1md
            val = lo + (c0.astype(jnp.float32) + descoff) * bw
            return acc + om * val, rank + incl[L - 1]

        acc, _ = rd_loop
        return jnp.sum(acc) * invw

    pltpu.async_copy(x_hbm.at[base], bufa, sem0)

    def pair_body(p, resvec):
        row0 = base + 2 * p
        pltpu.async_copy(x_hbm.at[row0 + 1], bufb, sem1)
        pltpu.make_async_copy(x_hbm.at[row0], bufa, sem0).wait()
        s0 = row_compute(bufa)

        @pl.when(2 * p + 2 < ROWS_PER_W)
        def _():
            pltpu.async_copy(x_hbm.at[row0 + 2], bufa, sem0)

        pltpu.make_async_copy(x_hbm.at[row0 + 1], bufb, sem1).wait()
        s1 = row_compute(bufb)

        r0 = (2 * p) % L
        resvec = (resvec + jnp.where(lanes == r0, s0, 0.0)
                  + jnp.where(lanes == r0 + 1, s1, 0.0))

        @pl.when((p % (L // 2)) == (L // 2 - 1))
        def _():
            outbuf[pl.ds((p - (L // 2 - 1)) * 2, L)] = resvec

        return jnp.where(p % (L // 2) == (L // 2 - 1),
                         jnp.zeros((L,), jnp.float32), resvec)

    lax.fori_loop(0, ROWS_PER_W // 2, pair_body,
                  jnp.zeros((L,), jnp.float32))
    pltpu.sync_copy(outbuf, out_hbm.at[pl.ds(base, ROWS_PER_W)])


def kernel(input, gwrp_w):
    x = input.reshape(NROWS, H, W)
    K = gwrp_w.shape[0]
    d = gwrp_w[1]
    lam = jnp.log(d)
    inv1md = 1.0 / (1.0 - d)
    invw = 1.0 / jnp.sum(gwrp_w)
    params = jnp.zeros((L,), jnp.float32)
    params = params.at[0].set(lam).at[1].set(inv1md).at[2].set(invw)

    mesh = plsc.VectorSubcoreMesh(core_axis_name="c", subcore_axis_name="s")
    run = pl.kernel(
        functools.partial(_sc_body, K),
        out_type=jax.ShapeDtypeStruct((NROWS,), jnp.float32),
        mesh=mesh,
        compiler_params=pltpu.CompilerParams(needs_layout_passes=False,
                                             use_tc_tiling_on_sc=True),
        scratch_types=[
            pltpu.VMEM((H, W), jnp.float32),
            pltpu.VMEM((H, W), jnp.float32),
            pltpu.VMEM((L * NB2,), jnp.float32),
            pltpu.VMEM((ROWS_PER_W,), jnp.float32),
            pltpu.VMEM((L,), jnp.float32),
            pltpu.SemaphoreType.DMA,
            pltpu.SemaphoreType.DMA,
        ],
    )
    out = run(x, params)
    return out.reshape(B, C)
